# baseline (speedup 1.0000x reference)
import jax
import jax.numpy as jnp
from jax import lax
from jax.experimental import pallas as pl
from jax.experimental.pallas import tpu as pltpu
from jax.experimental.pallas import tpu_sc as plsc

B = 16384
L = 32
NT = 4
W = L * NT
NW = 32
BPW = B // NW
CH = 128
NCH = BPW // CH
LANES = 16


def _sc_body(users, items, uE_all, iE_all,
             uB_a, uB_b, uB_g, uB_d,
             iB_a, iB_b, iB_g, iB_d,
             rp, d0, d1, d2, d3, d4, price, gb16,
             out,
             uidx, iidx, uidx2, iidx2,
             ubuf0, ibuf0, ubuf1, ibuf1,
             ub0, ub1, ub2, ub3, ib0, ib1, ib2, ib3,
             rpg, prg, dg0, dg1, dg2, dg3, dg4,
             fac0, fac1, fac2, fac3, gbv,
             sem_rows, sem_bias, sem_misc, sem_idx):
    wid = lax.axis_index("s") * 2 + lax.axis_index("c")
    base = wid * BPW

    idx_copies = [pltpu.async_copy(users.at[pl.ds(base, BPW)], uidx, sem_idx),
                  pltpu.async_copy(items.at[pl.ds(base, BPW)], iidx, sem_idx),
                  pltpu.async_copy(gb16, gbv, sem_idx)]
    for j in range(NCH):
        idx_copies.append(
            pltpu.async_copy(users.at[pl.ds(base + j * CH, CH)], uidx2.at[j], sem_idx))
        idx_copies.append(
            pltpu.async_copy(items.at[pl.ds(base + j * CH, CH)], iidx2.at[j], sem_idx))
    for c in idx_copies:
        c.wait()

    ubuf = [ubuf0, ubuf1]
    ibuf = [ibuf0, ibuf1]
    ubs = [ub0, ub1, ub2, ub3]
    ibs = [ib0, ib1, ib2, ib3]
    dgs = [dg0, dg1, dg2, dg3, dg4]
    dcols = [d0, d1, d2, d3, d4]
    facs = [fac0, fac1, fac2, fac3]

    row_copies = [None, None]
    row_copies[0] = (pltpu.async_copy(uE_all.at[uidx2.at[0]], ubuf[0], sem_rows),
                     pltpu.async_copy(iE_all.at[iidx2.at[0]], ibuf[0], sem_rows))

    bias_copies = [
        pltpu.async_copy(uB_a.at[uidx], ubs[0], sem_bias),
        pltpu.async_copy(uB_b.at[uidx], ubs[1], sem_bias),
        pltpu.async_copy(uB_g.at[uidx], ubs[2], sem_bias),
        pltpu.async_copy(uB_d.at[uidx], ubs[3], sem_bias),
        pltpu.async_copy(iB_a.at[iidx], ibs[0], sem_bias),
        pltpu.async_copy(iB_b.at[iidx], ibs[1], sem_bias),
        pltpu.async_copy(iB_g.at[iidx], ibs[2], sem_bias),
        pltpu.async_copy(iB_d.at[iidx], ibs[3], sem_bias),
    ]
    misc_copies = [pltpu.async_copy(rp.at[uidx], rpg, sem_misc),
                   pltpu.async_copy(price.at[iidx], prg, sem_misc)]
    for r in range(5):
        misc_copies.append(pltpu.async_copy(dcols[r].at[iidx], dgs[r], sem_misc))

    for c in bias_copies:
        c.wait()

    for j in range(NCH):
        buf = j % 2
        if j < NCH - 1:
            nbuf = (j + 1) % 2
            row_copies[nbuf] = (
                pltpu.async_copy(uE_all.at[uidx2.at[j + 1]], ubuf[nbuf], sem_rows),
                pltpu.async_copy(iE_all.at[iidx2.at[j + 1]], ibuf[nbuf], sem_rows))
        row_copies[buf][0].wait()
        row_copies[buf][1].wait()

        ur = ubuf[buf]
        ir = ibuf[buf]

        def _dot_body(g, _, ur=ur, ir=ir, j=j):
            rows = lax.iota(jnp.int32, LANES) + g * LANES
            off = j * CH + g * LANES
            for n in range(NT):
                accs = [gbv[n] + ubs[n][pl.ds(off, LANES)] + ibs[n][pl.ds(off, LANES)],
                        jnp.zeros((LANES,), jnp.float32),
                        jnp.zeros((LANES,), jnp.float32),
                        jnp.zeros((LANES,), jnp.float32)]
                for c in range(L):
                    cc = jnp.full((LANES,), n * L + c, jnp.int32)
                    uv = plsc.load_gather(ur, [rows, cc])
                    iv = plsc.load_gather(ir, [rows, cc])
                    accs[c % 4] = accs[c % 4] + uv * iv
                acc = (accs[0] + accs[1]) + (accs[2] + accs[3])
                facs[n][pl.ds(off, LANES)] = jnp.maximum(jnp.float32(0.1), acc)
            return 0
        lax.fori_loop(0, CH // LANES, _dot_body, 0)

    for n in range(NT):
        pltpu.sync_copy(facs[n], out.at[pl.ds(n * B + base, BPW)])

    for c in misc_copies:
        c.wait()
    pltpu.sync_copy(rpg, out.at[pl.ds(4 * B + base, BPW)])
    pltpu.sync_copy(prg, out.at[pl.ds(5 * B + base, BPW)])
    for r in range(5):
        pltpu.sync_copy(dgs[r], out.at[pl.ds((6 + r) * B + base, BPW)])


@jax.jit
def _sc_stage(users, items, uE_all, iE_all, uBs, iBs, rp, dcols, price, gb16):
    f32 = jnp.float32
    i32 = jnp.int32
    scratch = [
        pltpu.VMEM((BPW,), i32),
        pltpu.VMEM((BPW,), i32),
        pltpu.VMEM((NCH, CH), i32),
        pltpu.VMEM((NCH, CH), i32),
    ]
    scratch += [pltpu.VMEM((CH, W), f32) for _ in range(4)]
    scratch += [pltpu.VMEM((BPW,), f32) for _ in range(8)]
    scratch += [pltpu.VMEM((BPW,), f32) for _ in range(7)]
    scratch += [pltpu.VMEM((BPW,), f32) for _ in range(4)]
    scratch += [pltpu.VMEM((4, LANES), f32)]
    scratch += [pltpu.SemaphoreType.DMA] * 4

    mesh = plsc.VectorSubcoreMesh(core_axis_name="c", subcore_axis_name="s",
                                  num_cores=2, num_subcores=16)
    k = pl.kernel(
        _sc_body,
        out_type=jax.ShapeDtypeStruct((11 * B,), f32),
        mesh=mesh,
        scratch_types=scratch,
        compiler_params=pltpu.CompilerParams(needs_layout_passes=False,
                                             use_tc_tiling_on_sc=False),
    )
    return k(users, items, uE_all, iE_all, *uBs, *iBs, rp, *dcols, price, gb16)


def _pack_body(a_ref, b_ref, g_ref, d_ref, o_ref):
    row = lax.broadcasted_iota(jnp.int32, (L, L), 0)
    col = lax.broadcasted_iota(jnp.int32, (L, L), 1)
    eye = (row == col).astype(jnp.float32)
    dn = (((0,), (0,)), ((), ()))
    cols = [lax.dot_general(r[...], eye, dimension_numbers=dn,
                            precision=lax.Precision.HIGHEST,
                            preferred_element_type=jnp.float32)
            for r in (a_ref, b_ref, g_ref, d_ref)]
    o_ref[...] = jnp.concatenate(cols, axis=1)


def _pack_stage(uEt_list, n_rows):
    blk = 2048
    grid = (n_rows + blk - 1) // blk
    return pl.pallas_call(
        _pack_body,
        grid=(grid,),
        in_specs=[pl.BlockSpec((L, blk), lambda i: (0, i)) for _ in range(4)],
        out_specs=pl.BlockSpec((blk, W), lambda i: (i, 0)),
        out_shape=jax.ShapeDtypeStruct((n_rows, W), jnp.float32),
    )(*uEt_list)


def _tc_body(x_ref, o_ref):
    alpha = x_ref[0]
    beta = x_ref[1]
    gamma = x_ref[2]
    delta = x_ref[3]
    rp = x_ref[4]
    price = x_ref[5]
    inv_g = 1.0 / gamma
    inv_d = 1.0 / delta
    acc = jnp.zeros(alpha.shape, jnp.float32)
    for ri in range(5):
        r = jnp.float32(ri + 1)
        d = x_ref[6 + ri]
        tr = jnp.tanh(r - rp)
        pos = (tr > 0).astype(jnp.float32)
        neg = 1.0 - pos
        tp = pos * d
        tn = neg * d
        num_p = jnp.exp(gamma * jnp.log(tp))
        den_p = jnp.exp(inv_g * jnp.log(num_p + jnp.exp(gamma * jnp.log(1.0 - tp))))
        wp = num_p / den_p
        num_n = jnp.exp(delta * jnp.log(tn))
        den_n = jnp.exp(inv_d * jnp.log(num_n + jnp.exp(delta * jnp.log(1.0 - tn))))
        wn = num_n / den_n
        rpos = tr * pos
        rneg = tr * neg
        val = alpha * jnp.log(rpos * price + 1.0) - beta * jnp.log(1.0 - rneg * price)
        acc = acc + (wp + wn) * val
    o_ref[:] = acc


def _tc_stage(packed3):
    return pl.pallas_call(
        _tc_body,
        out_shape=jax.ShapeDtypeStruct(packed3.shape[1:], jnp.float32),
    )(packed3)


def kernel(users, items, gB_a, uB_a, iB_a, uE_a, iE_a, gB_b, uB_b, iB_b,
           uE_b, iE_b, gB_g, uB_g, iB_g, uE_g, iE_g, gB_d, uB_d, iB_d,
           uE_d, iE_d, reference_point, distribution, item_price):
    users32 = users.astype(jnp.int32)
    items32 = items.astype(jnp.int32)
    uE_all = _pack_stage([uE_a.T, uE_b.T, uE_g.T, uE_d.T], uE_a.shape[0])
    iE_all = _pack_stage([iE_a.T, iE_b.T, iE_g.T, iE_d.T], iE_a.shape[0])
    gb16 = jnp.stack([
        jnp.full((LANES,), gB_a, jnp.float32),
        jnp.full((LANES,), gB_b, jnp.float32),
        jnp.full((LANES,), gB_g, jnp.float32),
        jnp.full((LANES,), gB_d, jnp.float32),
    ])
    dcols = [distribution[:, r] for r in range(5)]
    packed = _sc_stage(
        users32, items32, uE_all, iE_all,
        (uB_a.reshape(-1), uB_b.reshape(-1), uB_g.reshape(-1), uB_d.reshape(-1)),
        (iB_a.reshape(-1), iB_b.reshape(-1), iB_g.reshape(-1), iB_d.reshape(-1)),
        reference_point.reshape(-1), dcols, item_price,
        gb16)
    out2d = _tc_stage(packed.reshape(11, 128, 128))
    return out2d.reshape(B)


# speedup vs baseline: 1.5423x; 1.5423x over previous
import jax
import jax.numpy as jnp
from jax import lax
from jax.experimental import pallas as pl
from jax.experimental.pallas import tpu as pltpu
from jax.experimental.pallas import tpu_sc as plsc

B = 16384
L = 32
NT = 4
W = L * NT
NW = 32
BPW = B // NW
CH = 128
NCH = BPW // CH
LANES = 16


def _sc_body(users, items, uE_all, iE_all,
             uB_a, uB_b, uB_g, uB_d,
             iB_a, iB_b, iB_g, iB_d,
             rp, d0, d1, d2, d3, d4, price, gb16,
             out,
             uidx, iidx, uidx2, iidx2,
             ubuf0, ibuf0, ubuf1, ibuf1,
             ub0, ub1, ub2, ub3, ib0, ib1, ib2, ib3,
             rpg, prg, dg0, dg1, dg2, dg3, dg4,
             fac0, fac1, fac2, fac3, gbv,
             sem_rows, sem_bias, sem_misc, sem_idx):
    wid = lax.axis_index("s") * 2 + lax.axis_index("c")
    base = wid * BPW

    idx_copies = [pltpu.async_copy(users.at[pl.ds(base, BPW)], uidx, sem_idx),
                  pltpu.async_copy(items.at[pl.ds(base, BPW)], iidx, sem_idx),
                  pltpu.async_copy(gb16, gbv, sem_idx)]
    for j in range(NCH):
        idx_copies.append(
            pltpu.async_copy(users.at[pl.ds(base + j * CH, CH)], uidx2.at[j], sem_idx))
        idx_copies.append(
            pltpu.async_copy(items.at[pl.ds(base + j * CH, CH)], iidx2.at[j], sem_idx))
    for c in idx_copies:
        c.wait()

    ubuf = [ubuf0, ubuf1]
    ibuf = [ibuf0, ibuf1]
    ubs = [ub0, ub1, ub2, ub3]
    ibs = [ib0, ib1, ib2, ib3]
    dgs = [dg0, dg1, dg2, dg3, dg4]
    dcols = [d0, d1, d2, d3, d4]
    facs = [fac0, fac1, fac2, fac3]

    row_copies = [None, None]
    row_copies[0] = (pltpu.async_copy(uE_all.at[uidx2.at[0]], ubuf[0], sem_rows),
                     pltpu.async_copy(iE_all.at[iidx2.at[0]], ibuf[0], sem_rows))

    bias_copies = [
        pltpu.async_copy(uB_a.at[uidx], ubs[0], sem_bias),
        pltpu.async_copy(uB_b.at[uidx], ubs[1], sem_bias),
        pltpu.async_copy(uB_g.at[uidx], ubs[2], sem_bias),
        pltpu.async_copy(uB_d.at[uidx], ubs[3], sem_bias),
        pltpu.async_copy(iB_a.at[iidx], ibs[0], sem_bias),
        pltpu.async_copy(iB_b.at[iidx], ibs[1], sem_bias),
        pltpu.async_copy(iB_g.at[iidx], ibs[2], sem_bias),
        pltpu.async_copy(iB_d.at[iidx], ibs[3], sem_bias),
    ]
    misc_copies = [pltpu.async_copy(rp.at[uidx], rpg, sem_misc),
                   pltpu.async_copy(price.at[iidx], prg, sem_misc)]
    for r in range(5):
        misc_copies.append(pltpu.async_copy(dcols[r].at[iidx], dgs[r], sem_misc))

    for c in bias_copies:
        c.wait()

    for j in range(NCH):
        buf = j % 2
        if j < NCH - 1:
            nbuf = (j + 1) % 2
            row_copies[nbuf] = (
                pltpu.async_copy(uE_all.at[uidx2.at[j + 1]], ubuf[nbuf], sem_rows),
                pltpu.async_copy(iE_all.at[iidx2.at[j + 1]], ibuf[nbuf], sem_rows))
        row_copies[buf][0].wait()
        row_copies[buf][1].wait()

        ur = ubuf[buf]
        ir = ibuf[buf]

        def _dot_body(g, _, ur=ur, ir=ir, j=j):
            rows = lax.iota(jnp.int32, LANES) + g * LANES
            off = j * CH + g * LANES
            for n in range(NT):
                accs = [gbv[n] + ubs[n][pl.ds(off, LANES)] + ibs[n][pl.ds(off, LANES)],
                        jnp.zeros((LANES,), jnp.float32),
                        jnp.zeros((LANES,), jnp.float32),
                        jnp.zeros((LANES,), jnp.float32)]
                for c in range(L):
                    cc = jnp.full((LANES,), n * L + c, jnp.int32)
                    uv = plsc.load_gather(ur, [rows, cc])
                    iv = plsc.load_gather(ir, [rows, cc])
                    accs[c % 4] = accs[c % 4] + uv * iv
                acc = (accs[0] + accs[1]) + (accs[2] + accs[3])
                facs[n][pl.ds(off, LANES)] = jnp.maximum(jnp.float32(0.1), acc)
            return 0
        lax.fori_loop(0, CH // LANES, _dot_body, 0)

    for n in range(NT):
        pltpu.sync_copy(facs[n], out.at[pl.ds(n * B + base, BPW)])

    for c in misc_copies:
        c.wait()
    pltpu.sync_copy(rpg, out.at[pl.ds(4 * B + base, BPW)])
    pltpu.sync_copy(prg, out.at[pl.ds(5 * B + base, BPW)])
    for r in range(5):
        pltpu.sync_copy(dgs[r], out.at[pl.ds((6 + r) * B + base, BPW)])


@jax.jit
def _sc_stage(users, items, uE_all, iE_all, uBs, iBs, rp, dcols, price, gb16):
    f32 = jnp.float32
    i32 = jnp.int32
    scratch = [
        pltpu.VMEM((BPW,), i32),
        pltpu.VMEM((BPW,), i32),
        pltpu.VMEM((NCH, CH), i32),
        pltpu.VMEM((NCH, CH), i32),
    ]
    scratch += [pltpu.VMEM((CH, W), f32) for _ in range(4)]
    scratch += [pltpu.VMEM((BPW,), f32) for _ in range(8)]
    scratch += [pltpu.VMEM((BPW,), f32) for _ in range(7)]
    scratch += [pltpu.VMEM((BPW,), f32) for _ in range(4)]
    scratch += [pltpu.VMEM((4, LANES), f32)]
    scratch += [pltpu.SemaphoreType.DMA] * 4

    mesh = plsc.VectorSubcoreMesh(core_axis_name="c", subcore_axis_name="s",
                                  num_cores=2, num_subcores=16)
    k = pl.kernel(
        _sc_body,
        out_type=jax.ShapeDtypeStruct((11 * B,), f32),
        mesh=mesh,
        scratch_types=scratch,
        compiler_params=pltpu.CompilerParams(needs_layout_passes=False,
                                             use_tc_tiling_on_sc=False),
    )
    return k(users, items, uE_all, iE_all, *uBs, *iBs, rp, *dcols, price, gb16)


def _tc_body(x_ref, o_ref):
    alpha = x_ref[0]
    beta = x_ref[1]
    gamma = x_ref[2]
    delta = x_ref[3]
    rp = x_ref[4]
    price = x_ref[5]
    inv_g = 1.0 / gamma
    inv_d = 1.0 / delta
    acc = jnp.zeros(alpha.shape, jnp.float32)
    for ri in range(5):
        r = jnp.float32(ri + 1)
        d = x_ref[6 + ri]
        tr = jnp.tanh(r - rp)
        pos = (tr > 0).astype(jnp.float32)
        neg = 1.0 - pos
        tp = pos * d
        tn = neg * d
        num_p = jnp.exp(gamma * jnp.log(tp))
        den_p = jnp.exp(inv_g * jnp.log(num_p + jnp.exp(gamma * jnp.log(1.0 - tp))))
        wp = num_p / den_p
        num_n = jnp.exp(delta * jnp.log(tn))
        den_n = jnp.exp(inv_d * jnp.log(num_n + jnp.exp(delta * jnp.log(1.0 - tn))))
        wn = num_n / den_n
        rpos = tr * pos
        rneg = tr * neg
        val = alpha * jnp.log(rpos * price + 1.0) - beta * jnp.log(1.0 - rneg * price)
        acc = acc + (wp + wn) * val
    o_ref[:] = acc


def _tc_stage(packed3):
    return pl.pallas_call(
        _tc_body,
        out_shape=jax.ShapeDtypeStruct(packed3.shape[1:], jnp.float32),
    )(packed3)


def kernel(users, items, gB_a, uB_a, iB_a, uE_a, iE_a, gB_b, uB_b, iB_b,
           uE_b, iE_b, gB_g, uB_g, iB_g, uE_g, iE_g, gB_d, uB_d, iB_d,
           uE_d, iE_d, reference_point, distribution, item_price):
    users32 = users.astype(jnp.int32)
    items32 = items.astype(jnp.int32)
    uE_all = jnp.concatenate([uE_a, uE_b, uE_g, uE_d], axis=1)
    iE_all = jnp.concatenate([iE_a, iE_b, iE_g, iE_d], axis=1)
    gb16 = jnp.stack([
        jnp.full((LANES,), gB_a, jnp.float32),
        jnp.full((LANES,), gB_b, jnp.float32),
        jnp.full((LANES,), gB_g, jnp.float32),
        jnp.full((LANES,), gB_d, jnp.float32),
    ])
    dcols = [distribution[:, r] for r in range(5)]
    packed = _sc_stage(
        users32, items32, uE_all, iE_all,
        (uB_a.reshape(-1), uB_b.reshape(-1), uB_g.reshape(-1), uB_d.reshape(-1)),
        (iB_a.reshape(-1), iB_b.reshape(-1), iB_g.reshape(-1), iB_d.reshape(-1)),
        reference_point.reshape(-1), dcols, item_price,
        gb16)
    out2d = _tc_stage(packed.reshape(11, 128, 128))
    return out2d.reshape(B)
